# baseline (speedup 1.0000x reference)
import jax, jax.numpy as jnp
from jax.experimental import pallas as pl

_NU, _NI, _D = 50000, 25000, 64

def _l2n(x):
    ss = jnp.sum(x * x, axis=1, keepdims=True)
    return x / jnp.sqrt(jnp.maximum(ss, 1e-24))

def _spmm(rows, cols, vals, x, n):
    return jax.ops.segment_sum(vals[:, None] * jnp.take(x, cols, axis=0), rows, num_segments=n)

def _dummy_body(x_ref, o_ref):
    o_ref[...] = x_ref[...]

def kernel(ui_src, ui_dst, ui_vals, image_feats, text_feats, user_emb, item_emb, W_img, b_img, W_txt, b_txt):
    pu = jnp.argsort(ui_src)
    su_r, su_c, su_v = ui_src[pu], ui_dst[pu], ui_vals[pu]
    pi = jnp.argsort(ui_dst)
    si_r, si_c, si_v = ui_dst[pi], ui_src[pi], ui_vals[pi]
    bounds_u = jnp.searchsorted(su_r, jnp.arange(0, 51200 + 1600, 1600))
    bounds_i = jnp.searchsorted(si_r, jnp.arange(0, 25600 + 800, 800))

    image_feat = image_feats @ W_img + b_img
    text_feat = text_feats @ W_txt + b_txt
    X1 = jnp.concatenate([image_feat, text_feat, item_emb], axis=1)
    U1 = _spmm(su_r, su_c, su_v, X1, _NU)
    I1 = _spmm(si_r, si_c, si_v, U1, _NI)
    img_u, txt_u, u_g1 = U1[:, :64], U1[:, 64:128], U1[:, 128:]
    img_i, txt_i, i_g1 = I1[:, :64], I1[:, 64:128], I1[:, 128:]
    u_g2 = jax.nn.softmax(_spmm(su_r, su_c, su_v, i_g1, _NU) + 0.0 * jnp.sum(bounds_u).astype(jnp.float32), axis=-1)
    i_g2 = jax.nn.softmax(_spmm(si_r, si_c, si_v, u_g2, _NI) + 0.0 * jnp.sum(bounds_i).astype(jnp.float32), axis=-1)
    ue = (user_emb + u_g1 + u_g2) / 3.0
    ie = (item_emb + i_g1 + i_g2) / 3.0
    u_out = ue + 0.55 * _l2n(img_u) + 0.55 * _l2n(txt_u)
    i_out = ie + 0.55 * _l2n(img_i) + 0.55 * _l2n(txt_i)
    u_out = pl.pallas_call(_dummy_body, out_shape=jax.ShapeDtypeStruct((_NU, _D), jnp.float32))(u_out)
    z_u = jnp.zeros((_NU, _D), jnp.float32)
    z_i = jnp.zeros((_NI, _D), jnp.float32)
    return (u_out, i_out, img_i, txt_i, img_u, txt_u, u_out, i_out, z_u, z_i)


# speedup vs baseline: 2.2167x; 2.2167x over previous
import jax
import jax.numpy as jnp
from jax import lax
from jax.experimental import pallas as pl
from jax.experimental.pallas import tpu as pltpu
from jax.experimental.pallas import tpu_sc as plsc

_NU = 50000
_NI = 25000
_D = 64
_IMG = 4096
_TXT = 1024
_CAT = 0.55

_NC, _NS = 2, 16
_NW = _NC * _NS
_E = 800000
_EW = 25600
_E_PAD = _NW * _EW
_CK = 128
_NCHUNK = _EW // _CK
_SUP = 2048
_RU = 1600
_NU_PAD = _NW * _RU
_NI_PAD = 25600


def _make_sorted_spmm(n_tables, ru):
    mesh = plsc.VectorSubcoreMesh(core_axis_name="c", subcore_axis_name="s")

    def body(rows_hbm, cols_hbm, vals_hbm, bounds_hbm, *rest):
        tables = rest[:n_tables]
        outs = rest[n_tables:2 * n_tables]
        bounds_v, rows_s, cols_s, vals_s, gbuf, gbuf2, acc, gsem, gsem2 = rest[2 * n_tables:]
        c = lax.axis_index("c")
        s = lax.axis_index("s")
        w = c * _NS + s
        base = w * ru

        pltpu.sync_copy(bounds_hbm.at[w], bounds_v)
        bv = bounds_v[pl.ds(0, 16)]
        lo = bv[0]
        hi = bv[1]
        start = (lo // _SUP) * _SUP
        nsup = (hi - start + _SUP - 1) // _SUP

        for ti in range(n_tables):
            table = tables[ti]

            def zero(i, _):
                for q in range(4):
                    acc[i, pl.ds(q * 16, 16)] = jnp.zeros((16,), jnp.float32)
                return 0
            lax.fori_loop(0, ru, zero, 0)

            def accum(kc, buf):
                def grp(g, _):
                    o = kc * _CK + g * 16
                    rows16 = rows_s[pl.ds(o, 16)]
                    vals16 = vals_s[pl.ds(o, 16)]
                    in_m = (rows16 >= base) & (rows16 < base + ru)
                    mval = jnp.where(in_m, vals16, 0.0)
                    loc = jnp.minimum(jnp.maximum(rows16 - base, 0), ru - 1)
                    eb = g * 16
                    for j in range(16):
                        lr = loc[j]
                        v = mval[j]
                        for q in range(4):
                            sl = pl.ds(q * 16, 16)
                            plsc.addupdate(acc.at[lr, sl], buf[eb + j, sl] * v)
                    return 0
                lax.fori_loop(0, _CK // 16, grp, 0)

            npair = _SUP // _CK // 2
            dummy = rows_hbm

            def sup(si, _):
                e0 = start + si * _SUP
                pltpu.sync_copy(rows_hbm.at[pl.ds(e0, _SUP)], rows_s)
                pltpu.sync_copy(cols_hbm.at[pl.ds(e0, _SUP)], cols_s)
                pltpu.sync_copy(vals_hbm.at[pl.ds(e0, _SUP)], vals_s)

                pltpu.async_copy(table.at[cols_s.at[pl.ds(0, _CK)]], gbuf, gsem)
                pltpu.async_copy(table.at[cols_s.at[pl.ds(_CK, _CK)]], gbuf2, gsem2)

                def pair(k, _):
                    k2 = 2 * k
                    pltpu.make_async_copy(table.at[pl.ds(0, _CK), :], gbuf, gsem).wait()
                    accum(k2, gbuf)

                    @pl.when(k < npair - 1)
                    def _():
                        pltpu.async_copy(
                            table.at[cols_s.at[pl.ds((k2 + 2) * _CK, _CK)]], gbuf, gsem)

                    pltpu.make_async_copy(table.at[pl.ds(0, _CK), :], gbuf2, gsem2).wait()
                    accum(k2 + 1, gbuf2)

                    @pl.when(k < npair - 1)
                    def _():
                        pltpu.async_copy(
                            table.at[cols_s.at[pl.ds((k2 + 3) * _CK, _CK)]], gbuf2, gsem2)
                    return 0
                lax.fori_loop(0, npair, pair, 0)
                return 0
            lax.fori_loop(0, nsup, sup, 0)

            pltpu.sync_copy(acc, outs[ti].at[pl.ds(base, ru), :])

    sds = jax.ShapeDtypeStruct((_NW * ru, _D), jnp.float32)
    return pl.kernel(
        body,
        out_type=[sds] * n_tables,
        mesh=mesh,
        scratch_types=[
            pltpu.VMEM((16,), jnp.int32),
            pltpu.VMEM((_SUP,), jnp.int32),
            pltpu.VMEM((_SUP,), jnp.int32),
            pltpu.VMEM((_SUP,), jnp.float32),
            pltpu.VMEM((_CK, _D), jnp.float32),
            pltpu.VMEM((_CK, _D), jnp.float32),
            pltpu.VMEM((ru, _D), jnp.float32),
            pltpu.SemaphoreType.DMA,
            pltpu.SemaphoreType.DMA,
        ],
        compiler_params=pltpu.CompilerParams(use_tc_tiling_on_sc=False),
    )


def _proj_body(img_ref, txt_ref, wi_ref, bi_ref, wt_ref, bt_ref, oi_ref, ot_ref):
    oi_ref[...] = jnp.dot(img_ref[...], wi_ref[...], preferred_element_type=jnp.float32) + bi_ref[...]
    ot_ref[...] = jnp.dot(txt_ref[...], wt_ref[...], preferred_element_type=jnp.float32) + bt_ref[...]


def _project(image_feats, text_feats, W_img, b_img, W_txt, b_txt):
    BN = 1000
    out_spec = pl.BlockSpec((BN, _D), lambda i: (i, 0))
    sds = jax.ShapeDtypeStruct((_NI, _D), jnp.float32)
    return pl.pallas_call(
        _proj_body,
        grid=(_NI // BN,),
        in_specs=[
            pl.BlockSpec((BN, _IMG), lambda i: (i, 0)),
            pl.BlockSpec((BN, _TXT), lambda i: (i, 0)),
            pl.BlockSpec((_IMG, _D), lambda i: (0, 0)),
            pl.BlockSpec((1, _D), lambda i: (0, 0)),
            pl.BlockSpec((_TXT, _D), lambda i: (0, 0)),
            pl.BlockSpec((1, _D), lambda i: (0, 0)),
        ],
        out_specs=[out_spec, out_spec],
        out_shape=[sds, sds],
    )(image_feats, text_feats, W_img, b_img[None, :], W_txt, b_txt[None, :])


def _combine3(partial, n, bn, np_, cw):
    pp = np_ // 3

    def body(part_ref, o1_ref, o2_ref, o3_ref):
        s = part_ref[0] + part_ref[1]
        for k, o_ref in enumerate((o1_ref, o2_ref, o3_ref)):
            o_ref[...] = jnp.concatenate([s[k * pp + q] for q in range(pp)], axis=1)

    out_spec = pl.BlockSpec((bn, _D), lambda i: (i, 0))
    sds = jax.ShapeDtypeStruct((n, _D), jnp.float32)
    return pl.pallas_call(
        body,
        grid=(n // bn,),
        in_specs=[pl.BlockSpec((_NC, np_, bn, cw), lambda i: (0, 0, i, 0))],
        out_specs=[out_spec] * 3,
        out_shape=[sds] * 3,
    )(partial)


def _softmax_rows(x, n, bn):
    def body(x_ref, o_ref):
        o_ref[...] = jax.nn.softmax(x_ref[...], axis=-1)

    spec = pl.BlockSpec((bn, _D), lambda i: (i, 0))
    return pl.pallas_call(
        body,
        grid=(n // bn,),
        in_specs=[spec],
        out_specs=spec,
        out_shape=jax.ShapeDtypeStruct((n, _D), jnp.float32),
    )(x)


def _l2n(x):
    ss = jnp.sum(x * x, axis=1, keepdims=True)
    return x / jnp.sqrt(jnp.maximum(ss, 1e-24))


def _fin_user(emb, g1, g2, imgf, txtf, n, bn):
    def body(emb_ref, g1_ref, g2_ref, imgf_ref, txtf_ref, out_ref):
        mean = (emb_ref[...] + g1_ref[...] + g2_ref[...]) * (1.0 / 3.0)
        out_ref[...] = mean + _CAT * _l2n(imgf_ref[...]) + _CAT * _l2n(txtf_ref[...])

    spec = pl.BlockSpec((bn, _D), lambda i: (i, 0))
    return pl.pallas_call(
        body,
        grid=(n // bn,),
        in_specs=[spec] * 5,
        out_specs=spec,
        out_shape=jax.ShapeDtypeStruct((n, _D), jnp.float32),
    )(emb, g1, g2, imgf, txtf)


def _fin_item(emb, g1, p4raw, imgf, txtf, n, bn):
    def body(emb_ref, g1_ref, p4_ref, imgf_ref, txtf_ref, out_ref):
        g2 = jax.nn.softmax(p4_ref[...], axis=-1)
        mean = (emb_ref[...] + g1_ref[...] + g2) * (1.0 / 3.0)
        out_ref[...] = mean + _CAT * _l2n(imgf_ref[...]) + _CAT * _l2n(txtf_ref[...])

    spec = pl.BlockSpec((bn, _D), lambda i: (i, 0))
    return pl.pallas_call(
        body,
        grid=(n // bn,),
        in_specs=[spec] * 5,
        out_specs=spec,
        out_shape=jax.ShapeDtypeStruct((n, _D), jnp.float32),
    )(emb, g1, p4raw, imgf, txtf)


_RI = 800


def kernel(ui_src, ui_dst, ui_vals, image_feats, text_feats, user_emb, item_emb, W_img, b_img, W_txt, b_txt):
    npad = _E_PAD - _E
    src32 = ui_src.astype(jnp.int32)
    dst32 = ui_dst.astype(jnp.int32)

    uk, uc, uv = jax.lax.sort((src32, dst32, ui_vals), num_keys=1)
    su_r = jnp.concatenate([uk, jnp.full((npad,), _NU_PAD - 1, jnp.int32)])
    su_c = jnp.concatenate([uc, jnp.zeros((npad,), jnp.int32)])
    su_v = jnp.concatenate([uv, jnp.zeros((npad,), jnp.float32)])
    thr_u = jnp.arange(0, _NU_PAD + _RU, _RU, dtype=jnp.int32)
    bu = jnp.sum(su_r[:, None] < thr_u[None, :], axis=0, dtype=jnp.int32)
    bounds_u = jnp.zeros((_NW, 16), jnp.int32).at[:, 0].set(bu[:-1]).at[:, 1].set(bu[1:])

    ik, ic, iv = jax.lax.sort((dst32, src32, ui_vals), num_keys=1)
    si_r = jnp.concatenate([ik, jnp.full((npad,), _NI_PAD - 1, jnp.int32)])
    si_c = jnp.concatenate([ic, jnp.zeros((npad,), jnp.int32)])
    si_v = jnp.concatenate([iv, jnp.zeros((npad,), jnp.float32)])
    thr_i = jnp.arange(0, _NI_PAD + _RI, _RI, dtype=jnp.int32)
    bi = jnp.sum(si_r[:, None] < thr_i[None, :], axis=0, dtype=jnp.int32)
    bounds_i = jnp.zeros((_NW, 16), jnp.int32).at[:, 0].set(bi[:-1]).at[:, 1].set(bi[1:])

    img_feat, txt_feat = _project(image_feats, text_feats, W_img, b_img, W_txt, b_txt)

    spmm_user3 = _make_sorted_spmm(3, _RU)
    img_u, txt_u, u_g1 = spmm_user3(su_r, su_c, su_v, bounds_u, img_feat, txt_feat, item_emb)

    spmm_item3 = _make_sorted_spmm(3, _RI)
    img_i, txt_i, i_g1 = spmm_item3(si_r, si_c, si_v, bounds_i, img_u, txt_u, u_g1)
    img_i, txt_i, i_g1 = img_i[:_NI], txt_i[:_NI], i_g1[:_NI]

    spmm_user1 = _make_sorted_spmm(1, _RU)
    (p3raw,) = spmm_user1(su_r, su_c, su_v, bounds_u, i_g1)
    u_g2 = _softmax_rows(p3raw[:_NU], _NU, 400)

    spmm_item1 = _make_sorted_spmm(1, _RI)
    (p4raw,) = spmm_item1(si_r, si_c, si_v, bounds_i, u_g2)

    img_u, txt_u, u_g1 = img_u[:_NU], txt_u[:_NU], u_g1[:_NU]
    u_out = _fin_user(user_emb, u_g1, u_g2, img_u, txt_u, _NU, 400)
    i_out = _fin_item(item_emb, i_g1, p4raw[:_NI], img_i, txt_i, _NI, 200)

    pu = jnp.zeros((_NU, _D), jnp.float32)
    pi = jnp.zeros((_NI, _D), jnp.float32)
    return (u_out, i_out, img_i, txt_i, img_u, txt_u, u_out, i_out, pu, pi)
